# baseline (speedup 1.0000x reference)
import functools

import jax
import jax.numpy as jnp
from jax import lax
from jax.experimental import pallas as pl
from jax.experimental.pallas import tpu as pltpu
from jax.experimental.pallas import tpu_sc as plsc

N = 10000
E = 320000
D_IN = 128
D_EDGE = 16
D_OUT = 128
D_U = 2 * D_OUT

NC = 2
NS = 16
NW = NC * NS
EPW = E // NW

KS = 80
NCS = EPW // KS
KE = 80
NCE = EPW // KE

NPAD = 10240
RPT = NPAD // NS
RC = 80
NRC = RPT // RC

LANES = 16
CPR = D_OUT // LANES


def _mesh():
    return plsc.VectorSubcoreMesh(
        core_axis_name="c", subcore_axis_name="s", num_cores=NC, num_subcores=NS
    )


def _p_body(x_ref, w_ref, b_ref, o_ref):
    o_ref[...] = (
        jnp.dot(x_ref[...], w_ref[...], preferred_element_type=jnp.float32)
        + b_ref[...]
    )


def _node_proj(node_feats, wn_t, b2):
    return pl.pallas_call(
        _p_body,
        grid=(10,),
        in_specs=[
            pl.BlockSpec((N // 10, D_IN), lambda i: (i, 0)),
            pl.BlockSpec((D_IN, D_OUT), lambda i: (0, 0)),
            pl.BlockSpec((1, D_OUT), lambda i: (0, 0)),
        ],
        out_specs=pl.BlockSpec((N // 10, D_OUT), lambda i: (i, 0)),
        out_shape=jax.ShapeDtypeStruct((N, D_OUT), jnp.float32),
    )(node_feats, wn_t, b2)


def _q_body(x_ref, w_ref, o_ref):
    o_ref[...] = jnp.dot(x_ref[...], w_ref[...], preferred_element_type=jnp.float32)


def _edge_proj(edge_feats, we_t):
    blk = 4000
    return pl.pallas_call(
        _q_body,
        grid=(E // blk,),
        in_specs=[
            pl.BlockSpec((blk, D_EDGE), lambda i: (i, 0)),
            pl.BlockSpec((D_EDGE, D_OUT), lambda i: (0, 0)),
        ],
        out_specs=pl.BlockSpec((blk, D_OUT), lambda i: (i, 0)),
        out_shape=jax.ShapeDtypeStruct((E, D_OUT), jnp.float32),
    )(edge_feats, we_t)


def _comb_body(t0_ref, t1_ref, pn_ref, nn_ref, u_ref):
    f = -(t0_ref[...] + t1_ref[...])
    nn_ref[...] = jnp.maximum(f, 0.0)
    u_ref[:, :D_OUT] = f
    u_ref[:, D_OUT:] = pn_ref[...]


def _combine(t01, pn):
    blk = 80
    return pl.pallas_call(
        _comb_body,
        grid=(N // blk,),
        in_specs=[
            pl.BlockSpec((blk, D_OUT), lambda i: (i, 0)),
            pl.BlockSpec((blk, D_OUT), lambda i: (i + NPAD // blk, 0)),
            pl.BlockSpec((blk, D_OUT), lambda i: (i, 0)),
        ],
        out_specs=[
            pl.BlockSpec((blk, D_OUT), lambda i: (i, 0)),
            pl.BlockSpec((blk, D_U), lambda i: (i, 0)),
        ],
        out_shape=[
            jax.ShapeDtypeStruct((N, D_OUT), jnp.float32),
            jax.ShapeDtypeStruct((N, D_U), jnp.float32),
        ],
    )(t01, t01, pn)


def _scatter_kernel(
    src2_hbm, dst_hbm, q_hbm, pn_hbm, t_out,
    tsh, sidx_v, dst_v0, dst_v1, st_v0, st_v1, qn_v, gsem0, gsem1, qsem,
):
    cid = lax.axis_index("c")
    sid = lax.axis_index("s")
    wid = sid * NC + cid
    dst_v = (dst_v0, dst_v1)
    st_v = (st_v0, st_v1)
    gsem = (gsem0, gsem1)

    pltpu.sync_copy(src2_hbm.at[wid], sidx_v)

    def zrow(r, carry):
        for cc in range(CPR):
            st_v0[r, pl.ds(cc * LANES, LANES)] = jnp.zeros((LANES,), jnp.float32)
        return carry

    lax.fori_loop(0, RC, zrow, 0)
    for j in range(NRC):
        r0 = sid * RPT + j * RC
        pltpu.sync_copy(st_v0, tsh.at[pl.ds(r0, RC)])
    plsc.subcore_barrier()

    def fire(i, s):
        base = wid * EPW + i * KS
        pltpu.async_copy(dst_hbm.at[pl.ds(base, KS)], dst_v[s], gsem[s])
        pltpu.async_copy(
            pn_hbm.at[sidx_v.at[pl.ds(i * KS, KS)]], st_v[s], gsem[s]
        )

    def fire_qn(i):
        base = wid * EPW + i * KS
        pltpu.async_copy(q_hbm.at[pl.ds(base, KS)], qn_v, qsem)

    def consume(i, s):
        pltpu.make_async_copy(dst_hbm.at[pl.ds(0, KS)], dst_v[s], gsem[s]).wait()
        pltpu.make_async_copy(
            pn_hbm.at[sidx_v.at[pl.ds(0, KS)]], st_v[s], gsem[s]
        ).wait()
        pltpu.make_async_copy(q_hbm.at[pl.ds(0, KS)], qn_v, qsem).wait()
        st = st_v[s]

        def row4(r4, carry):
            r0 = r4 * 4
            for c in range(CPR):
                sl = pl.ds(c * LANES, LANES)
                for k in range(4):
                    r = r0 + k
                    plsc.addupdate(st.at[r, sl], qn_v[r, sl])
            return carry

        lax.fori_loop(0, KS // 4, row4, 0)
        fire_qn(jnp.minimum(i + 1, NCS - 1))
        pltpu.sync_copy(st, tsh.at[dst_v[s]], add=True)

    fire(0, 0)
    fire_qn(0)
    fire(1, 1)

    def pair(g, carry):
        i0 = 2 * g
        consume(i0, 0)
        fire(i0 + 2, 0)
        consume(i0 + 1, 1)
        fire(i0 + 3, 1)
        return carry

    lax.fori_loop(0, (NCS - 3) // 2, pair, 0)
    consume(NCS - 3, 0)
    fire(NCS - 1, 0)
    consume(NCS - 2, 1)
    consume(NCS - 1, 0)
    plsc.subcore_barrier()

    for j in range(NRC):
        r0 = sid * RPT + j * RC
        pltpu.sync_copy(tsh.at[pl.ds(r0, RC)], st_v0)
        pltpu.sync_copy(st_v0, t_out.at[pl.ds(cid * NPAD + r0, RC)])


def _segment_sums(src2, dst, qn, pn):
    k = functools.partial(
        pl.kernel,
        out_type=jax.ShapeDtypeStruct((NC * NPAD, D_OUT), jnp.float32),
        mesh=_mesh(),
        scratch_types=[
            pltpu.VMEM_SHARED((NPAD, D_OUT), jnp.float32),
            pltpu.VMEM((EPW,), jnp.int32),
            pltpu.VMEM((KS,), jnp.int32),
            pltpu.VMEM((KS,), jnp.int32),
            pltpu.VMEM((KS, D_OUT), jnp.float32),
            pltpu.VMEM((KS, D_OUT), jnp.float32),
            pltpu.VMEM((KS, D_OUT), jnp.float32),
            pltpu.SemaphoreType.DMA,
            pltpu.SemaphoreType.DMA,
            pltpu.SemaphoreType.DMA,
        ],
    )(_scatter_kernel)
    return k(src2, dst, qn, pn)


def _edge_out_kernel(
    src2_hbm, dst2_hbm, q_hbm, u_hbm, nd_hbm, nb_hbm,
    sidx_v, didx_v, us_v0, us_v1, ud_v0, ud_v1, q_v0, q_v1,
    gsem0, gsem1, ssem0, ssem1,
):
    cid = lax.axis_index("c")
    sid = lax.axis_index("s")
    wid = sid * NC + cid
    us_v = (us_v0, us_v1)
    ud_v = (ud_v0, ud_v1)
    q_v = (q_v0, q_v1)
    gsem = (gsem0, gsem1)
    ssem = (ssem0, ssem1)

    pltpu.sync_copy(src2_hbm.at[wid], sidx_v)
    pltpu.sync_copy(dst2_hbm.at[wid], didx_v)

    def fire_gathers(i, s):
        base = wid * EPW + i * KE
        pltpu.async_copy(q_hbm.at[pl.ds(base, KE)], q_v[s], gsem[s])
        pltpu.async_copy(u_hbm.at[sidx_v.at[pl.ds(i * KE, KE)]], us_v[s], gsem[s])
        pltpu.async_copy(u_hbm.at[didx_v.at[pl.ds(i * KE, KE)]], ud_v[s], gsem[s])

    def wait_gathers(s):
        pltpu.make_async_copy(q_hbm.at[pl.ds(0, KE)], q_v[s], gsem[s]).wait()
        pltpu.make_async_copy(
            u_hbm.at[sidx_v.at[pl.ds(0, KE)]], us_v[s], gsem[s]
        ).wait()
        pltpu.make_async_copy(
            u_hbm.at[didx_v.at[pl.ds(0, KE)]], ud_v[s], gsem[s]
        ).wait()

    def compute(s):
        us, ud, qq = us_v[s], ud_v[s], q_v[s]

        def row4(r4, carry):
            r0 = r4 * 4
            for c in range(CPR):
                sl = pl.ds(c * LANES, LANES)
                sl2 = pl.ds(D_OUT + c * LANES, LANES)
                for k in range(4):
                    r = r0 + k
                    qn = qq[r, sl]
                    plsc.addupdate(us.at[r, sl], qn + ud[r, sl2])
                    plsc.addupdate(ud.at[r, sl], qn + us[r, sl2])
            return carry

        lax.fori_loop(0, KE // 4, row4, 0)

    def fire_stores(i, s):
        base = wid * EPW + i * KE
        pltpu.async_copy(
            us_v[s].at[:, pl.ds(0, D_OUT)], nd_hbm.at[pl.ds(base, KE)], ssem[s]
        )
        pltpu.async_copy(
            ud_v[s].at[:, pl.ds(0, D_OUT)], nb_hbm.at[pl.ds(base, KE)], ssem[s]
        )

    def wait_stores(s):
        pltpu.make_async_copy(
            us_v[s].at[:, pl.ds(0, D_OUT)], nd_hbm.at[pl.ds(0, KE)], ssem[s]
        ).wait()
        pltpu.make_async_copy(
            ud_v[s].at[:, pl.ds(0, D_OUT)], nb_hbm.at[pl.ds(0, KE)], ssem[s]
        ).wait()

    fire_gathers(0, 0)
    fire_gathers(1, 1)
    wait_gathers(0)
    compute(0)
    fire_stores(0, 0)
    wait_stores(0)
    fire_gathers(2, 0)
    wait_gathers(1)
    compute(1)
    fire_stores(1, 1)

    def pair(g, carry):
        i0 = 2 * g + 2
        wait_stores(1)
        fire_gathers(i0 + 1, 1)
        wait_gathers(0)
        compute(0)
        fire_stores(i0, 0)
        wait_stores(0)
        fire_gathers(i0 + 2, 0)
        wait_gathers(1)
        compute(1)
        fire_stores(i0 + 1, 1)
        return carry

    lax.fori_loop(0, (NCE - 3) // 2, pair, 0)
    wait_gathers(0)
    compute(0)
    fire_stores(NCE - 1, 0)
    wait_stores(0)
    wait_stores(1)


def _edge_outputs(src2, dst2, qn, u):
    k = functools.partial(
        pl.kernel,
        out_type=(
            jax.ShapeDtypeStruct((E, D_OUT), jnp.float32),
            jax.ShapeDtypeStruct((E, D_OUT), jnp.float32),
        ),
        mesh=_mesh(),
        scratch_types=[
            pltpu.VMEM((EPW,), jnp.int32),
            pltpu.VMEM((EPW,), jnp.int32),
            pltpu.VMEM((KE, D_U), jnp.float32),
            pltpu.VMEM((KE, D_U), jnp.float32),
            pltpu.VMEM((KE, D_U), jnp.float32),
            pltpu.VMEM((KE, D_U), jnp.float32),
            pltpu.VMEM((KE, D_OUT), jnp.float32),
            pltpu.VMEM((KE, D_OUT), jnp.float32),
            pltpu.SemaphoreType.DMA,
            pltpu.SemaphoreType.DMA,
            pltpu.SemaphoreType.DMA,
            pltpu.SemaphoreType.DMA,
        ],
    )(_edge_out_kernel)
    return k(src2, dst2, qn, u)


def kernel(node_feats, edge_index, edge_feats, W, b):
    src = edge_index[0]
    dst = edge_index[1]
    src2 = src.reshape(NW, EPW)
    dst2 = dst.reshape(NW, EPW)
    we_tn = -W[:, :D_EDGE].T
    wn_tn = -W[:, D_EDGE:].T
    bn2 = (-b).reshape(1, D_OUT)

    pn = _node_proj(node_feats, wn_tn, bn2)
    qn = _edge_proj(edge_feats, we_tn)
    t01 = _segment_sums(src2, dst, qn, pn)
    new_node, u = _combine(t01, pn)
    new_direct, new_backward = _edge_outputs(src2, dst2, qn, u)
    return (new_node, new_direct, new_backward)


# speedup vs baseline: 1.3816x; 1.3816x over previous
import functools

import jax
import jax.numpy as jnp
from jax import lax
from jax.experimental import pallas as pl
from jax.experimental.pallas import tpu as pltpu
from jax.experimental.pallas import tpu_sc as plsc

N = 10000
E = 320000
D_IN = 128
D_EDGE = 16
D_OUT = 128
D_U = 2 * D_OUT

NC = 2
NS = 16
NW = NC * NS
EPW = E // NW

KS = 80
NCS = EPW // KS
KE = 80
NCE = EPW // KE

NPAD = 10240
RPT = NPAD // NS
RC = 80
NRC = RPT // RC

LANES = 16
CPR = D_OUT // LANES


def _mesh():
    return plsc.VectorSubcoreMesh(
        core_axis_name="c", subcore_axis_name="s", num_cores=NC, num_subcores=NS
    )


PBLK = N // 10
QBLK = E // 80


def _proj_body(x_ref, wn_ref, b_ref, ef_ref, we_ref, pn_ref, qn_ref):
    i = pl.program_id(0)

    @pl.when(i < 10)
    def _():
        pn_ref[...] = (
            jnp.dot(x_ref[...], wn_ref[...], preferred_element_type=jnp.float32)
            + b_ref[...]
        )

    @pl.when(i >= 10)
    def _():
        qn_ref[...] = jnp.dot(
            ef_ref[...], we_ref[...], preferred_element_type=jnp.float32
        )


def _projections(node_feats, wn_t, b2, edge_feats, we_t):
    return pl.pallas_call(
        _proj_body,
        grid=(90,),
        in_specs=[
            pl.BlockSpec((PBLK, D_IN), lambda i: (jnp.minimum(i, 9), 0)),
            pl.BlockSpec((D_IN, D_OUT), lambda i: (0, 0)),
            pl.BlockSpec((1, D_OUT), lambda i: (0, 0)),
            pl.BlockSpec((QBLK, D_EDGE), lambda i: (jnp.maximum(i - 10, 0), 0)),
            pl.BlockSpec((D_EDGE, D_OUT), lambda i: (0, 0)),
        ],
        out_specs=[
            pl.BlockSpec((PBLK, D_OUT), lambda i: (jnp.minimum(i, 9), 0)),
            pl.BlockSpec((QBLK, D_OUT), lambda i: (jnp.maximum(i - 10, 0), 0)),
        ],
        out_shape=[
            jax.ShapeDtypeStruct((N, D_OUT), jnp.float32),
            jax.ShapeDtypeStruct((E, D_OUT), jnp.float32),
        ],
    )(node_feats, wn_t, b2, edge_feats, we_t)


def _comb_body(t0_ref, t1_ref, pn_ref, nn_ref, u_ref):
    f = -(t0_ref[...] + t1_ref[...])
    nn_ref[...] = jnp.maximum(f, 0.0)
    u_ref[:, :D_OUT] = f
    u_ref[:, D_OUT:] = pn_ref[...]


def _combine(t01, pn):
    blk = 80
    return pl.pallas_call(
        _comb_body,
        grid=(N // blk,),
        in_specs=[
            pl.BlockSpec((blk, D_OUT), lambda i: (i, 0)),
            pl.BlockSpec((blk, D_OUT), lambda i: (i + NPAD // blk, 0)),
            pl.BlockSpec((blk, D_OUT), lambda i: (i, 0)),
        ],
        out_specs=[
            pl.BlockSpec((blk, D_OUT), lambda i: (i, 0)),
            pl.BlockSpec((blk, D_U), lambda i: (i, 0)),
        ],
        out_shape=[
            jax.ShapeDtypeStruct((N, D_OUT), jnp.float32),
            jax.ShapeDtypeStruct((N, D_U), jnp.float32),
        ],
    )(t01, t01, pn)


def _scatter_kernel(
    src2_hbm, dst_hbm, q_hbm, pn_hbm, t_out,
    tsh, sidx_v, dst_v0, dst_v1, st_v0, st_v1, qn_v, gsem0, gsem1, qsem,
):
    cid = lax.axis_index("c")
    sid = lax.axis_index("s")
    wid = sid * NC + cid
    dst_v = (dst_v0, dst_v1)
    st_v = (st_v0, st_v1)
    gsem = (gsem0, gsem1)

    pltpu.sync_copy(src2_hbm.at[wid], sidx_v)

    def zrow(r, carry):
        for cc in range(CPR):
            st_v0[r, pl.ds(cc * LANES, LANES)] = jnp.zeros((LANES,), jnp.float32)
        return carry

    lax.fori_loop(0, RC, zrow, 0)
    for j in range(NRC):
        r0 = sid * RPT + j * RC
        pltpu.sync_copy(st_v0, tsh.at[pl.ds(r0, RC)])
    plsc.subcore_barrier()

    def fire(i, s):
        base = wid * EPW + i * KS
        pltpu.async_copy(dst_hbm.at[pl.ds(base, KS)], dst_v[s], gsem[s])
        pltpu.async_copy(
            pn_hbm.at[sidx_v.at[pl.ds(i * KS, KS)]], st_v[s], gsem[s]
        )

    def fire_qn(i):
        base = wid * EPW + i * KS
        pltpu.async_copy(q_hbm.at[pl.ds(base, KS)], qn_v, qsem)

    def consume(i, s):
        pltpu.make_async_copy(dst_hbm.at[pl.ds(0, KS)], dst_v[s], gsem[s]).wait()
        pltpu.make_async_copy(
            pn_hbm.at[sidx_v.at[pl.ds(0, KS)]], st_v[s], gsem[s]
        ).wait()
        pltpu.make_async_copy(q_hbm.at[pl.ds(0, KS)], qn_v, qsem).wait()
        st = st_v[s]

        def row(r, carry):
            for c in range(CPR):
                sl = pl.ds(c * LANES, LANES)
                plsc.addupdate(st.at[r, sl], qn_v[r, sl])
            return carry

        lax.fori_loop(0, KS, row, 0)
        fire_qn(jnp.minimum(i + 1, NCS - 1))
        pltpu.sync_copy(st, tsh.at[dst_v[s]], add=True)

    fire(0, 0)
    fire_qn(0)
    fire(1, 1)

    def pair(g, carry):
        i0 = 2 * g
        consume(i0, 0)
        fire(i0 + 2, 0)
        consume(i0 + 1, 1)
        fire(i0 + 3, 1)
        return carry

    lax.fori_loop(0, (NCS - 3) // 2, pair, 0)
    consume(NCS - 3, 0)
    fire(NCS - 1, 0)
    consume(NCS - 2, 1)
    consume(NCS - 1, 0)
    plsc.subcore_barrier()

    for j in range(NRC):
        r0 = sid * RPT + j * RC
        pltpu.sync_copy(tsh.at[pl.ds(r0, RC)], st_v0)
        pltpu.sync_copy(st_v0, t_out.at[pl.ds(cid * NPAD + r0, RC)])


def _segment_sums(src2, dst, qn, pn):
    k = functools.partial(
        pl.kernel,
        out_type=jax.ShapeDtypeStruct((NC * NPAD, D_OUT), jnp.float32),
        mesh=_mesh(),
        scratch_types=[
            pltpu.VMEM_SHARED((NPAD, D_OUT), jnp.float32),
            pltpu.VMEM((EPW,), jnp.int32),
            pltpu.VMEM((KS,), jnp.int32),
            pltpu.VMEM((KS,), jnp.int32),
            pltpu.VMEM((KS, D_OUT), jnp.float32),
            pltpu.VMEM((KS, D_OUT), jnp.float32),
            pltpu.VMEM((KS, D_OUT), jnp.float32),
            pltpu.SemaphoreType.DMA,
            pltpu.SemaphoreType.DMA,
            pltpu.SemaphoreType.DMA,
        ],
    )(_scatter_kernel)
    return k(src2, dst, qn, pn)


def _edge_out_kernel(
    src2_hbm, dst2_hbm, q_hbm, u_hbm, nd_hbm, nb_hbm,
    sidx_v, didx_v, us_v0, us_v1, ud_v0, ud_v1, q_v0, q_v1,
    gsem0, gsem1, ssem0, ssem1,
):
    cid = lax.axis_index("c")
    sid = lax.axis_index("s")
    wid = sid * NC + cid
    us_v = (us_v0, us_v1)
    ud_v = (ud_v0, ud_v1)
    q_v = (q_v0, q_v1)
    gsem = (gsem0, gsem1)
    ssem = (ssem0, ssem1)

    pltpu.sync_copy(src2_hbm.at[wid], sidx_v)
    pltpu.sync_copy(dst2_hbm.at[wid], didx_v)

    def fire_gathers(i, s):
        base = wid * EPW + i * KE
        pltpu.async_copy(q_hbm.at[pl.ds(base, KE)], q_v[s], gsem[s])
        pltpu.async_copy(u_hbm.at[sidx_v.at[pl.ds(i * KE, KE)]], us_v[s], gsem[s])
        pltpu.async_copy(u_hbm.at[didx_v.at[pl.ds(i * KE, KE)]], ud_v[s], gsem[s])

    def wait_gathers(s):
        pltpu.make_async_copy(q_hbm.at[pl.ds(0, KE)], q_v[s], gsem[s]).wait()
        pltpu.make_async_copy(
            u_hbm.at[sidx_v.at[pl.ds(0, KE)]], us_v[s], gsem[s]
        ).wait()
        pltpu.make_async_copy(
            u_hbm.at[didx_v.at[pl.ds(0, KE)]], ud_v[s], gsem[s]
        ).wait()

    def compute(s):
        us, ud, qq = us_v[s], ud_v[s], q_v[s]

        def row(r, carry):
            for c in range(CPR):
                sl = pl.ds(c * LANES, LANES)
                sl2 = pl.ds(D_OUT + c * LANES, LANES)
                qn = qq[r, sl]
                plsc.addupdate(us.at[r, sl], qn + ud[r, sl2])
                plsc.addupdate(ud.at[r, sl], qn + us[r, sl2])
            return carry

        lax.fori_loop(0, KE, row, 0)

    def fire_stores(i, s):
        base = wid * EPW + i * KE
        pltpu.async_copy(
            us_v[s].at[:, pl.ds(0, D_OUT)], nd_hbm.at[pl.ds(base, KE)], ssem[s]
        )
        pltpu.async_copy(
            ud_v[s].at[:, pl.ds(0, D_OUT)], nb_hbm.at[pl.ds(base, KE)], ssem[s]
        )

    def wait_stores(s):
        pltpu.make_async_copy(
            us_v[s].at[:, pl.ds(0, D_OUT)], nd_hbm.at[pl.ds(0, KE)], ssem[s]
        ).wait()
        pltpu.make_async_copy(
            ud_v[s].at[:, pl.ds(0, D_OUT)], nb_hbm.at[pl.ds(0, KE)], ssem[s]
        ).wait()

    fire_gathers(0, 0)
    fire_gathers(1, 1)
    wait_gathers(0)
    compute(0)
    fire_stores(0, 0)
    wait_stores(0)
    fire_gathers(2, 0)
    wait_gathers(1)
    compute(1)
    fire_stores(1, 1)

    def pair(g, carry):
        i0 = 2 * g + 2
        wait_stores(1)
        fire_gathers(i0 + 1, 1)
        wait_gathers(0)
        compute(0)
        fire_stores(i0, 0)
        wait_stores(0)
        fire_gathers(i0 + 2, 0)
        wait_gathers(1)
        compute(1)
        fire_stores(i0 + 1, 1)
        return carry

    lax.fori_loop(0, (NCE - 3) // 2, pair, 0)
    wait_gathers(0)
    compute(0)
    fire_stores(NCE - 1, 0)
    wait_stores(0)
    wait_stores(1)


def _edge_outputs(src2, dst2, qn, u):
    k = functools.partial(
        pl.kernel,
        out_type=(
            jax.ShapeDtypeStruct((E, D_OUT), jnp.float32),
            jax.ShapeDtypeStruct((E, D_OUT), jnp.float32),
        ),
        mesh=_mesh(),
        scratch_types=[
            pltpu.VMEM((EPW,), jnp.int32),
            pltpu.VMEM((EPW,), jnp.int32),
            pltpu.VMEM((KE, D_U), jnp.float32),
            pltpu.VMEM((KE, D_U), jnp.float32),
            pltpu.VMEM((KE, D_U), jnp.float32),
            pltpu.VMEM((KE, D_U), jnp.float32),
            pltpu.VMEM((KE, D_OUT), jnp.float32),
            pltpu.VMEM((KE, D_OUT), jnp.float32),
            pltpu.SemaphoreType.DMA,
            pltpu.SemaphoreType.DMA,
            pltpu.SemaphoreType.DMA,
            pltpu.SemaphoreType.DMA,
        ],
    )(_edge_out_kernel)
    return k(src2, dst2, qn, u)


def kernel(node_feats, edge_index, edge_feats, W, b):
    src = edge_index[0]
    dst = edge_index[1]
    src2 = src.reshape(NW, EPW)
    dst2 = dst.reshape(NW, EPW)
    we_tn = -W[:, :D_EDGE].T
    wn_tn = -W[:, D_EDGE:].T
    bn2 = (-b).reshape(1, D_OUT)

    pn, qn = _projections(node_feats, wn_tn, bn2, edge_feats, we_tn)
    t01 = _segment_sums(src2, dst, qn, pn)
    new_node, u = _combine(t01, pn)
    new_direct, new_backward = _edge_outputs(src2, dst2, qn, u)
    return (new_node, new_direct, new_backward)
